# baseline (device time: 104570 ns/iter reference)
import jax
import jax.numpy as jnp
from jax import lax
from jax.experimental import pallas as pl
from jax.experimental.pallas import tpu as pltpu

N_DEV = 16
LANES = 8
SLOTS = 3
COMM_DTYPE = jnp.bfloat16

_sem_signal = getattr(pl, "semaphore_signal", None) or pltpu.semaphore_signal
_sem_wait = getattr(pl, "semaphore_wait", None) or pltpu.semaphore_wait
_CompilerParams = getattr(pltpu, "CompilerParams", None) or pltpu.TPUCompilerParams


def kernel(x, w_mat):
    M, k_per = x.shape
    _, N = w_mat.shape
    Mc = M // N_DEV
    Nl = N // LANES

    def body(x_ref, w_ref, out_ref, *scratch):
        sbufs = scratch[0:LANES]
        rbufs = scratch[LANES:2 * LANES]
        amax_ref = scratch[2 * LANES]
        amax_recv = scratch[2 * LANES + 1]
        ssems = scratch[2 * LANES + 2:3 * LANES + 2]
        rsems = scratch[3 * LANES + 2:4 * LANES + 2]
        asend_sems = scratch[4 * LANES + 2]
        arecv_sems = scratch[4 * LANES + 3]
        creds = scratch[4 * LANES + 4:5 * LANES + 4]

        my = lax.axis_index("i")
        left = lax.rem(my - 1 + N_DEV, N_DEV)
        right = lax.rem(my + 1, N_DEV)

        lanes = []
        for i in range(LANES):
            goes_right = (i % 2 == 0)
            col = (i // 2) * Nl if goes_right else (LANES // 2 + i // 2) * Nl
            dst, src = (right, left) if goes_right else (left, right)
            lanes.append((sbufs[i], rbufs[i], ssems[i], rsems[i], creds[i],
                          dst, src, col, goes_right))

        barrier_sem = pltpu.get_barrier_semaphore()
        for nbr in (left, right):
            _sem_signal(barrier_sem, inc=1, device_id=(nbr,),
                        device_id_type=pl.DeviceIdType.MESH)
        _sem_wait(barrier_sem, 2)

        def rdma(lane, slot):
            sb, rb, ss, rs_, _, dst, src, col, goes_right = lanes[lane]
            return pltpu.make_async_remote_copy(
                src_ref=sb.at[slot], dst_ref=rb.at[slot],
                send_sem=ss.at[slot], recv_sem=rs_.at[slot],
                device_id=(dst,), device_id_type=pl.DeviceIdType.MESH)

        for s in range(N_DEV):
            cR = lax.rem(my - 1 - s + 2 * N_DEV, N_DEV)
            cL = lax.rem(my + 1 + s, N_DEV)
            xR = x_ref[pl.ds(cR * Mc, Mc), :]
            xL = x_ref[pl.ds(cL * Mc, Mc), :]
            for lane in range(LANES):
                sb, rb, ss, rs_, cred, dst, src, col, goes_right = lanes[lane]
                xc = xR if goes_right else xL
                p = jnp.dot(xc, w_ref[:, col:col + Nl],
                            preferred_element_type=jnp.float32)
                if s == 0:
                    acc = p
                else:
                    rslot = (s - 1) % SLOTS
                    rdma(lane, rslot).wait_recv()
                    acc = p + rb[rslot, :, :].astype(jnp.float32)
                    if s <= N_DEV - 1 - SLOTS:
                        _sem_signal(cred, inc=1, device_id=(src,),
                                    device_id_type=pl.DeviceIdType.MESH)
                if s < N_DEV - 1:
                    sslot = s % SLOTS
                    if s >= SLOTS:
                        rdma(lane, sslot).wait_send()
                        _sem_wait(cred, 1)
                    sb[sslot, :, :] = acc.astype(COMM_DTYPE)
                    rdma(lane, sslot).start()
                else:
                    out_ref[:, col:col + Nl] = acc

        for lane in range(LANES):
            for s in range(N_DEV - 1 - SLOTS, N_DEV - 1):
                rdma(lane, s % SLOTS).wait_send()

        local_amax = jnp.max(jnp.abs(out_ref[:, :]))
        amax_ref[:, :] = jnp.full((8, 128), local_amax, jnp.float32)
        for j in range(1, N_DEV):
            partner = my ^ j
            pltpu.make_async_remote_copy(
                src_ref=amax_ref, dst_ref=amax_recv.at[j],
                send_sem=asend_sems.at[j], recv_sem=arecv_sems.at[j],
                device_id=(partner,),
                device_id_type=pl.DeviceIdType.MESH).start()
        gmax = amax_ref[:, :]
        for j in range(1, N_DEV):
            partner = my ^ j
            pltpu.make_async_remote_copy(
                src_ref=amax_ref, dst_ref=amax_recv.at[j],
                send_sem=asend_sems.at[j], recv_sem=arecv_sems.at[j],
                device_id=(partner,),
                device_id_type=pl.DeviceIdType.MESH).wait_recv()
            gmax = jnp.maximum(gmax, amax_recv[j, :, :])
        for j in range(1, N_DEV):
            partner = my ^ j
            pltpu.make_async_remote_copy(
                src_ref=amax_ref, dst_ref=amax_recv.at[j],
                send_sem=asend_sems.at[j], recv_sem=arecv_sems.at[j],
                device_id=(partner,),
                device_id_type=pl.DeviceIdType.MESH).wait_send()

        scale = jnp.max(gmax) / 127.0
        q = jnp.clip(jnp.round(out_ref[:, :] / scale), -127.0, 127.0)
        out_ref[:, :] = q * scale

    return pl.pallas_call(
        body,
        out_shape=jax.ShapeDtypeStruct((Mc, N), jnp.float32),
        in_specs=[pl.BlockSpec(memory_space=pltpu.VMEM),
                  pl.BlockSpec(memory_space=pltpu.VMEM)],
        out_specs=pl.BlockSpec(memory_space=pltpu.VMEM),
        scratch_shapes=(
            [pltpu.VMEM((SLOTS, Mc, Nl), COMM_DTYPE)] * (2 * LANES) +
            [pltpu.VMEM((8, 128), jnp.float32),
             pltpu.VMEM((N_DEV, 8, 128), jnp.float32)] +
            [pltpu.SemaphoreType.DMA((SLOTS,))] * (2 * LANES) +
            [pltpu.SemaphoreType.DMA((N_DEV,)),
             pltpu.SemaphoreType.DMA((N_DEV,))] +
            [pltpu.SemaphoreType.REGULAR] * LANES
        ),
        compiler_params=_CompilerParams(collective_id=0),
    )(x, w_mat)


# device time: 104010 ns/iter; 1.0054x vs baseline; 1.0054x over previous
import jax
import jax.numpy as jnp
from jax import lax
from jax.experimental import pallas as pl
from jax.experimental.pallas import tpu as pltpu

N_DEV = 16
LANES = 4
SLOTS = 3
COMM_DTYPE = jnp.bfloat16

_sem_signal = getattr(pl, "semaphore_signal", None) or pltpu.semaphore_signal
_sem_wait = getattr(pl, "semaphore_wait", None) or pltpu.semaphore_wait
_CompilerParams = getattr(pltpu, "CompilerParams", None) or pltpu.TPUCompilerParams


def kernel(x, w_mat):
    M, k_per = x.shape
    _, N = w_mat.shape
    Mc = M // N_DEV
    Nl = N // LANES

    def body(x_ref, w_ref, out_ref, *scratch):
        sbufs = scratch[0:LANES]
        rbufs = scratch[LANES:2 * LANES]
        amax_ref = scratch[2 * LANES]
        amax_recv = scratch[2 * LANES + 1]
        ssems = scratch[2 * LANES + 2:3 * LANES + 2]
        rsems = scratch[3 * LANES + 2:4 * LANES + 2]
        asend_sems = scratch[4 * LANES + 2]
        arecv_sems = scratch[4 * LANES + 3]
        creds = scratch[4 * LANES + 4:5 * LANES + 4]

        my = lax.axis_index("i")
        left = lax.rem(my - 1 + N_DEV, N_DEV)
        right = lax.rem(my + 1, N_DEV)

        lanes = []
        for i in range(LANES):
            goes_right = (i % 2 == 0)
            col = (i // 2) * Nl if goes_right else (LANES // 2 + i // 2) * Nl
            dst, src = (right, left) if goes_right else (left, right)
            lanes.append((sbufs[i], rbufs[i], ssems[i], rsems[i], creds[i],
                          dst, src, col, goes_right))

        barrier_sem = pltpu.get_barrier_semaphore()
        for nbr in (left, right):
            _sem_signal(barrier_sem, inc=1, device_id=(nbr,),
                        device_id_type=pl.DeviceIdType.MESH)
        _sem_wait(barrier_sem, 2)

        def rdma(lane, slot):
            sb, rb, ss, rs_, _, dst, src, col, goes_right = lanes[lane]
            return pltpu.make_async_remote_copy(
                src_ref=sb.at[slot], dst_ref=rb.at[slot],
                send_sem=ss.at[slot], recv_sem=rs_.at[slot],
                device_id=(dst,), device_id_type=pl.DeviceIdType.MESH)

        lane_maxes = []
        for s in range(N_DEV):
            cR = lax.rem(my - 1 - s + 2 * N_DEV, N_DEV)
            cL = lax.rem(my + 1 + s, N_DEV)
            xR = x_ref[pl.ds(cR * Mc, Mc), :]
            xL = x_ref[pl.ds(cL * Mc, Mc), :]
            for lane in range(LANES):
                sb, rb, ss, rs_, cred, dst, src, col, goes_right = lanes[lane]
                xc = xR if goes_right else xL
                p = jnp.dot(xc, w_ref[:, col:col + Nl],
                            preferred_element_type=jnp.float32)
                if s == 0:
                    acc = p
                else:
                    rslot = (s - 1) % SLOTS
                    rdma(lane, rslot).wait_recv()
                    acc = p + rb[rslot, :, :].astype(jnp.float32)
                    if s <= N_DEV - 1 - SLOTS:
                        _sem_signal(cred, inc=1, device_id=(src,),
                                    device_id_type=pl.DeviceIdType.MESH)
                if s < N_DEV - 1:
                    sslot = s % SLOTS
                    if s >= SLOTS:
                        rdma(lane, sslot).wait_send()
                        _sem_wait(cred, 1)
                    sb[sslot, :, :] = acc.astype(COMM_DTYPE)
                    rdma(lane, sslot).start()
                else:
                    out_ref[:, col:col + Nl] = acc
                    lane_maxes.append(jnp.max(jnp.abs(acc)))

        local_amax = lane_maxes[0]
        for lm in lane_maxes[1:]:
            local_amax = jnp.maximum(local_amax, lm)
        amax_ref[:, :] = jnp.full((8, 128), local_amax, jnp.float32)
        for j in range(1, N_DEV):
            partner = my ^ j
            pltpu.make_async_remote_copy(
                src_ref=amax_ref, dst_ref=amax_recv.at[j],
                send_sem=asend_sems.at[j], recv_sem=arecv_sems.at[j],
                device_id=(partner,),
                device_id_type=pl.DeviceIdType.MESH).start()
        for lane in range(LANES):
            for s in range(N_DEV - 1 - SLOTS, N_DEV - 1):
                rdma(lane, s % SLOTS).wait_send()
        gmax = amax_ref[:, :]
        for j in range(1, N_DEV):
            partner = my ^ j
            pltpu.make_async_remote_copy(
                src_ref=amax_ref, dst_ref=amax_recv.at[j],
                send_sem=asend_sems.at[j], recv_sem=arecv_sems.at[j],
                device_id=(partner,),
                device_id_type=pl.DeviceIdType.MESH).wait_recv()
            gmax = jnp.maximum(gmax, amax_recv[j, :, :])
        for j in range(1, N_DEV):
            partner = my ^ j
            pltpu.make_async_remote_copy(
                src_ref=amax_ref, dst_ref=amax_recv.at[j],
                send_sem=asend_sems.at[j], recv_sem=arecv_sems.at[j],
                device_id=(partner,),
                device_id_type=pl.DeviceIdType.MESH).wait_send()

        scale = jnp.max(gmax) / 127.0
        q = jnp.clip(jnp.round(out_ref[:, :] / scale), -127.0, 127.0)
        out_ref[:, :] = q * scale

    return pl.pallas_call(
        body,
        out_shape=jax.ShapeDtypeStruct((Mc, N), jnp.float32),
        in_specs=[pl.BlockSpec(memory_space=pltpu.VMEM),
                  pl.BlockSpec(memory_space=pltpu.VMEM)],
        out_specs=pl.BlockSpec(memory_space=pltpu.VMEM),
        scratch_shapes=(
            [pltpu.VMEM((SLOTS, Mc, Nl), COMM_DTYPE)] * (2 * LANES) +
            [pltpu.VMEM((8, 128), jnp.float32),
             pltpu.VMEM((N_DEV, 8, 128), jnp.float32)] +
            [pltpu.SemaphoreType.DMA((SLOTS,))] * (2 * LANES) +
            [pltpu.SemaphoreType.DMA((N_DEV,)),
             pltpu.SemaphoreType.DMA((N_DEV,))] +
            [pltpu.SemaphoreType.REGULAR] * LANES
        ),
        compiler_params=_CompilerParams(collective_id=0),
    )(x, w_mat)
